# baseline (speedup 1.0000x reference)
import functools

import jax
import jax.numpy as jnp
from jax import lax
from jax.experimental import pallas as pl

REL_LIST = ["pp", "fp", "ep", "pf", "ef", "ff", "pe", "fe"]
REL_SRC_DST = {
    "pp": ("p", "p"), "fp": ("f", "p"), "ep": ("e", "p"),
    "pf": ("p", "f"), "ef": ("e", "f"), "ff": ("f", "f"),
    "pe": ("p", "e"), "fe": ("f", "e"),
}


def _enc_body(x_ref, w1t_ref, b1_ref, w2t_ref, b2_ref, o_ref):
    h = jnp.maximum(
        jnp.dot(x_ref[...], w1t_ref[...], preferred_element_type=jnp.float32)
        + b1_ref[...], 0.0)
    o_ref[...] = (
        jnp.dot(h, w2t_ref[...], preferred_element_type=jnp.float32)
        + b2_ref[...])


def _encoder_mlp(p, x, blk=512):
    n, in_dim = x.shape
    hid = p["W1"].shape[0]
    out_dim = p["W2"].shape[0]
    n_pad = ((n + blk - 1) // blk) * blk
    if n_pad != n:
        x = jnp.pad(x, ((0, n_pad - n), (0, 0)))
    out = pl.pallas_call(
        _enc_body,
        grid=(n_pad // blk,),
        in_specs=[
            pl.BlockSpec((blk, in_dim), lambda i: (i, 0)),
            pl.BlockSpec((in_dim, hid), lambda i: (0, 0)),
            pl.BlockSpec((1, hid), lambda i: (0, 0)),
            pl.BlockSpec((hid, out_dim), lambda i: (0, 0)),
            pl.BlockSpec((1, out_dim), lambda i: (0, 0)),
        ],
        out_specs=pl.BlockSpec((blk, out_dim), lambda i: (i, 0)),
        out_shape=jax.ShapeDtypeStruct((n_pad, out_dim), jnp.float32),
    )(x, p["W1"].T, p["b1"][None, :], p["W2"].T, p["b2"][None, :])
    return out[:n]


def _seg_sum_rows(vals, idx, n_seg):
    return jax.ops.segment_sum(vals, idx, num_segments=n_seg)


def _layer_norm(n, x):
    m = x.mean(-1, keepdims=True)
    v = jnp.mean((x - m) ** 2, axis=-1, keepdims=True)
    return (x - m) / jnp.sqrt(v + 1e-5) * n["g"] + n["b"]


def kernel(point_x, edge_x, face_x, ei_pp, ei_fp, ei_ep, ei_pf, ei_ef, ei_ff,
           ei_pe, ei_fe, batch_point, batch_edge, batch_face, params):
    ei = {"pp": ei_pp, "fp": ei_fp, "ep": ei_ep, "pf": ei_pf, "ef": ei_ef,
          "ff": ei_ff, "pe": ei_pe, "fe": ei_fe}
    h = {
        "p": _encoder_mlp(params["enc_point"], point_x),
        "e": _encoder_mlp(params["enc_edge"], edge_x),
        "f": _encoder_mlp(params["enc_face"], face_x),
    }
    n_nodes = {"p": point_x.shape[0], "e": edge_x.shape[0],
               "f": face_x.shape[0]}
    tname = {"p": "point", "e": "edge", "f": "face"}

    inv_cnt = {}
    for r in REL_LIST:
        dst = ei[r][1]
        n_dst = n_nodes[REL_SRC_DST[r][1]]
        cnt = jax.ops.segment_sum(
            jnp.ones((dst.shape[0],), jnp.float32), dst, num_segments=n_dst)
        inv_cnt[r] = 1.0 / jnp.maximum(cnt, 1.0)

    for i in range(2):
        cp = params["convs"][i]
        out = {}
        for t in ["p", "f", "e"]:
            rels = [r for r in REL_LIST if REL_SRC_DST[r][1] == t]
            wr_sum = sum(cp[r]["Wr"] for r in rels)
            bl_sum = sum(cp[r]["bl"] for r in rels)
            acc = h[t] @ wr_sum.T + bl_sum
            for r in rels:
                s_t = REL_SRC_DST[r][0]
                T = h[s_t] @ cp[r]["Wl"].T
                src, dst = ei[r][0], ei[r][1]
                agg = _seg_sum_rows(T[src], dst, n_nodes[t])
                acc = acc + agg * inv_cnt[r][:, None]
            out[t] = acc
        nrm = params["norms"][i]
        for t in ["p", "f", "e"]:
            h[t] = _layer_norm(nrm[tname[t]], h[t] + jnp.maximum(out[t], 0.0))

    def _pool(x, batch):
        s = jax.ops.segment_sum(x, batch, num_segments=16)
        c = jax.ops.segment_sum(
            jnp.ones((x.shape[0],), jnp.float32), batch, num_segments=16)
        return s / jnp.maximum(c, 1.0)[:, None]

    g_p = _pool(h["p"], batch_point)
    g_f = _pool(h["f"], batch_face)
    g_e = _pool(h["e"], batch_edge)
    return jnp.concatenate([g_p, g_e, g_f], axis=-1)


# speedup vs baseline: 2.7694x; 2.7694x over previous
import functools

import jax
import jax.numpy as jnp
from jax import lax
from jax.experimental import pallas as pl
from jax.experimental.pallas import tpu as pltpu
from jax.experimental.pallas import tpu_sc as plsc

REL_LIST = ["pp", "fp", "ep", "pf", "ef", "ff", "pe", "fe"]
REL_SRC_DST = {
    "pp": ("p", "p"), "fp": ("f", "p"), "ep": ("e", "p"),
    "pf": ("p", "f"), "ef": ("e", "f"), "ff": ("f", "f"),
    "pe": ("p", "e"), "fe": ("f", "e"),
}

HID = 64
BLK = 128
OUT_ROWS = 25088
TRASH = 128
ACC_ROWS = OUT_ROWS + TRASH
PER_TILE_ACC = ACC_ROWS // 16
PER_TILE_OUT = OUT_ROWS // 16
PAD_DST = 1 << 29

_MESH = dict(core_axis_name="c", subcore_axis_name="s", num_cores=2,
             num_subcores=16)
_SC_PARAMS = pltpu.CompilerParams(use_tc_tiling_on_sc=False)


def _compute_local_idx(dst_v, idx_v, base, trash0):
    for j in range(BLK // 16):
        d = dst_v[pl.ds(j * 16, 16)]
        local = d - base
        ok = (local >= 0) & (local < OUT_ROWS)
        idx_v[pl.ds(j * 16, 16)] = jnp.where(ok, local, trash0 + j)


def _make_agg(n_src, e_pad, split_edges):
    n_tiles = 32 if split_edges else 16
    chunk = e_pad // n_tiles
    n_blk = chunk // BLK
    mesh = plsc.VectorSubcoreMesh(**_MESH)

    @functools.partial(
        pl.kernel,
        out_type=jax.ShapeDtypeStruct((2, OUT_ROWS, HID), jnp.float32),
        mesh=mesh,
        compiler_params=_SC_PARAMS,
        scratch_types=[
            pltpu.VMEM((BLK,), jnp.int32),
            pltpu.VMEM((BLK,), jnp.int32),
            pltpu.VMEM((BLK,), jnp.int32),
            pltpu.VMEM((BLK, HID), jnp.float32),
            pltpu.VMEM_SHARED((ACC_ROWS, HID), jnp.float32),
            pltpu.SemaphoreType.DMA,
        ],
    )
    def agg(t_hbm, src_hbm, dst_hbm, out_hbm, src_v, dst_v, idx_v, rows_v,
            acc_sh, sem):
        cid = lax.axis_index("c")
        sid = lax.axis_index("s")

        def zrow(r, _):
            for k in range(HID // 16):
                rows_v[r, pl.ds(k * 16, 16)] = jnp.zeros((16,), jnp.float32)
            return 0
        lax.fori_loop(0, BLK, zrow, 0)
        r0 = sid * PER_TILE_ACC
        def zacc(k, _):
            pltpu.sync_copy(rows_v,
                            acc_sh.at[pl.ds(r0 + k * BLK, BLK), :])
            return 0
        lax.fori_loop(0, PER_TILE_ACC // BLK, zacc, 0)
        rem = PER_TILE_ACC % BLK
        if rem:
            pltpu.sync_copy(
                rows_v.at[pl.ds(0, rem), :],
                acc_sh.at[pl.ds(r0 + PER_TILE_ACC - rem, rem), :])
        plsc.subcore_barrier()

        if split_edges:
            base = jnp.int32(0)
            tid = cid * 16 + sid
        else:
            base = cid * OUT_ROWS
            tid = sid
        e_start = tid * chunk
        trash0 = OUT_ROWS + sid * (BLK // 16)

        def blk_body(b, _):
            e0 = e_start + b * BLK
            pltpu.sync_copy(src_hbm.at[pl.ds(e0, BLK)], src_v)
            pltpu.sync_copy(dst_hbm.at[pl.ds(e0, BLK)], dst_v)
            _compute_local_idx(dst_v, idx_v, base, trash0)
            pltpu.async_copy(t_hbm.at[src_v], rows_v, sem).wait()
            pltpu.sync_copy(rows_v, acc_sh.at[idx_v], add=True)
            return 0
        lax.fori_loop(0, n_blk, blk_body, 0)
        plsc.subcore_barrier()

        o0 = sid * PER_TILE_OUT
        pltpu.sync_copy(acc_sh.at[pl.ds(o0, PER_TILE_OUT), :],
                        out_hbm.at[cid, pl.ds(o0, PER_TILE_OUT), :])

    return agg


def _make_count(e_pad, split_edges):
    n_tiles = 32 if split_edges else 16
    chunk = e_pad // n_tiles
    n_blk = chunk // BLK
    mesh = plsc.VectorSubcoreMesh(**_MESH)

    @functools.partial(
        pl.kernel,
        out_type=jax.ShapeDtypeStruct((2 * OUT_ROWS,), jnp.float32),
        mesh=mesh,
        compiler_params=_SC_PARAMS,
        scratch_types=[
            pltpu.VMEM((BLK,), jnp.int32),
            pltpu.VMEM((BLK,), jnp.int32),
            pltpu.VMEM((BLK,), jnp.float32),
            pltpu.VMEM((PER_TILE_ACC,), jnp.float32),
            pltpu.VMEM_SHARED((ACC_ROWS,), jnp.float32),
        ],
    )
    def count(dst_hbm, out_hbm, dst_v, idx_v, ones_v, zflat, cnt_sh):
        cid = lax.axis_index("c")
        sid = lax.axis_index("s")

        def zf(i, _):
            zflat[pl.ds(i * 16, 16)] = jnp.zeros((16,), jnp.float32)
            return 0
        lax.fori_loop(0, PER_TILE_ACC // 16, zf, 0)
        if PER_TILE_ACC % 16:
            zflat[pl.ds(PER_TILE_ACC - 16, 16)] = jnp.zeros((16,), jnp.float32)
        for j in range(BLK // 16):
            ones_v[pl.ds(j * 16, 16)] = jnp.ones((16,), jnp.float32)
        pltpu.sync_copy(zflat, cnt_sh.at[pl.ds(sid * PER_TILE_ACC,
                                               PER_TILE_ACC)])
        plsc.subcore_barrier()

        if split_edges:
            base = jnp.int32(0)
            tid = cid * 16 + sid
        else:
            base = cid * OUT_ROWS
            tid = sid
        e_start = tid * chunk
        trash0 = OUT_ROWS + sid * (BLK // 16)

        def blk_body(b, _):
            e0 = e_start + b * BLK
            pltpu.sync_copy(dst_hbm.at[pl.ds(e0, BLK)], dst_v)
            _compute_local_idx(dst_v, idx_v, base, trash0)
            pltpu.sync_copy(ones_v, cnt_sh.at[idx_v], add=True)
            return 0
        lax.fori_loop(0, n_blk, blk_body, 0)
        plsc.subcore_barrier()

        o0 = sid * PER_TILE_OUT
        pltpu.sync_copy(cnt_sh.at[pl.ds(o0, PER_TILE_OUT)],
                        zflat.at[pl.ds(0, PER_TILE_OUT)])
        pltpu.sync_copy(zflat.at[pl.ds(0, PER_TILE_OUT)],
                        out_hbm.at[pl.ds(cid * OUT_ROWS + o0, PER_TILE_OUT)])

    return count


def _pad_edges(ei, split_edges):
    e = ei.shape[1]
    mult = BLK * (32 if split_edges else 16)
    e_pad = ((e + mult - 1) // mult) * mult
    src = jnp.concatenate(
        [ei[0], jnp.zeros((e_pad - e,), jnp.int32)])
    dst = jnp.concatenate(
        [ei[1], jnp.full((e_pad - e,), PAD_DST, jnp.int32)])
    return src, dst, e_pad


def _assemble(parts, idx_bound, n_dst, split_edges):
    if split_edges:
        full = parts[0] + parts[1]
        full = full[:min(idx_bound, n_dst)]
    else:
        full = jnp.concatenate([parts[0], parts[1]], axis=0)[:n_dst]
    if full.shape[0] < n_dst:
        pad = [(0, n_dst - full.shape[0])] + [(0, 0)] * (full.ndim - 1)
        full = jnp.pad(full, pad)
    return full


def _enc_body(x_ref, w1t_ref, b1_ref, w2t_ref, b2_ref, o_ref):
    h = jnp.maximum(
        jnp.dot(x_ref[...], w1t_ref[...], preferred_element_type=jnp.float32)
        + b1_ref[...], 0.0)
    o_ref[...] = (
        jnp.dot(h, w2t_ref[...], preferred_element_type=jnp.float32)
        + b2_ref[...])


def _encoder_mlp(p, x, blk=512):
    n, in_dim = x.shape
    hid = p["W1"].shape[0]
    out_dim = p["W2"].shape[0]
    n_pad = ((n + blk - 1) // blk) * blk
    if n_pad != n:
        x = jnp.pad(x, ((0, n_pad - n), (0, 0)))
    out = pl.pallas_call(
        _enc_body,
        grid=(n_pad // blk,),
        in_specs=[
            pl.BlockSpec((blk, in_dim), lambda i: (i, 0)),
            pl.BlockSpec((in_dim, hid), lambda i: (0, 0)),
            pl.BlockSpec((1, hid), lambda i: (0, 0)),
            pl.BlockSpec((hid, out_dim), lambda i: (0, 0)),
            pl.BlockSpec((1, out_dim), lambda i: (0, 0)),
        ],
        out_specs=pl.BlockSpec((blk, out_dim), lambda i: (i, 0)),
        out_shape=jax.ShapeDtypeStruct((n_pad, out_dim), jnp.float32),
    )(x, p["W1"].T, p["b1"][None, :], p["W2"].T, p["b2"][None, :])
    return out[:n]


def _layer_norm(n, x):
    m = x.mean(-1, keepdims=True)
    v = jnp.mean((x - m) ** 2, axis=-1, keepdims=True)
    return (x - m) / jnp.sqrt(v + 1e-5) * n["g"] + n["b"]


def kernel(point_x, edge_x, face_x, ei_pp, ei_fp, ei_ep, ei_pf, ei_ef, ei_ff,
           ei_pe, ei_fe, batch_point, batch_edge, batch_face, params):
    ei = {"pp": ei_pp, "fp": ei_fp, "ep": ei_ep, "pf": ei_pf, "ef": ei_ef,
          "ff": ei_ff, "pe": ei_pe, "fe": ei_fe}
    h = {
        "p": _encoder_mlp(params["enc_point"], point_x),
        "e": _encoder_mlp(params["enc_edge"], edge_x),
        "f": _encoder_mlp(params["enc_face"], face_x),
    }
    n_nodes = {"p": point_x.shape[0], "e": edge_x.shape[0],
               "f": face_x.shape[0]}
    tname = {"p": "point", "e": "edge", "f": "face"}

    prep, inv_cnt = {}, {}
    for r in REL_LIST:
        s_t, d_t = REL_SRC_DST[r]
        idx_bound = min(n_nodes[s_t], n_nodes[d_t])
        split_edges = idx_bound <= OUT_ROWS
        src, dst, e_pad = _pad_edges(ei[r], split_edges)
        prep[r] = (src, dst, e_pad, split_edges, idx_bound)
        cnt2 = _make_count(e_pad, split_edges)(dst).reshape(2, OUT_ROWS)
        cnt = _assemble(cnt2, idx_bound, n_nodes[d_t], split_edges)
        inv_cnt[r] = 1.0 / jnp.maximum(cnt, 1.0)

    for i in range(2):
        cp = params["convs"][i]
        out = {}
        for t in ["p", "f", "e"]:
            rels = [r for r in REL_LIST if REL_SRC_DST[r][1] == t]
            wr_sum = sum(cp[r]["Wr"] for r in rels)
            bl_sum = sum(cp[r]["bl"] for r in rels)
            acc = h[t] @ wr_sum.T + bl_sum
            for r in rels:
                s_t = REL_SRC_DST[r][0]
                src, dst, e_pad, split_edges, idx_bound = prep[r]
                T = h[s_t] @ cp[r]["Wl"].T
                agg2 = _make_agg(n_nodes[s_t], e_pad, split_edges)(T, src, dst)
                agg = _assemble(agg2, idx_bound, n_nodes[t], split_edges)
                acc = acc + agg * inv_cnt[r][:, None]
            out[t] = acc
        nrm = params["norms"][i]
        for t in ["p", "f", "e"]:
            h[t] = _layer_norm(nrm[tname[t]], h[t] + jnp.maximum(out[t], 0.0))

    def _pool(x, batch):
        s = jax.ops.segment_sum(x, batch, num_segments=16)
        c = jax.ops.segment_sum(
            jnp.ones((x.shape[0],), jnp.float32), batch, num_segments=16)
        return s / jnp.maximum(c, 1.0)[:, None]

    g_p = _pool(h["p"], batch_point)
    g_f = _pool(h["f"], batch_face)
    g_e = _pool(h["e"], batch_edge)
    return jnp.concatenate([g_p, g_e, g_f], axis=-1)
